# Initial kernel scaffold; baseline (speedup 1.0000x reference)
import jax
import jax.numpy as jnp
from jax.experimental import pallas as pl


def kernel(x):
    raise NotImplementedError("write your pallas kernel here")


# speedup vs baseline: 4.5445x; 4.5445x over previous
import functools

import jax
import jax.numpy as jnp
from jax import lax
from jax.experimental import pallas as pl
from jax.experimental.pallas import tpu as pltpu
from jax.experimental.pallas import tpu_sc as plsc

NUM_CLASSES = 12
NC = 2
NS = 16
L = 16
NW = NC * NS


@functools.lru_cache(maxsize=None)
def _make_onehot(total: int):
    assert total % NW == 0
    per_tile = total // NW
    chunk = 2048
    while per_tile % chunk:
        chunk //= 2
    assert chunk % L == 0
    n_chunks = per_tile // chunk
    out_c = chunk * NUM_CLASSES

    mesh = plsc.VectorSubcoreMesh(core_axis_name="c", subcore_axis_name="s")

    @functools.partial(
        pl.kernel,
        out_type=jax.ShapeDtypeStruct((total * NUM_CLASSES,), jnp.float32),
        mesh=mesh,
        compiler_params=pltpu.CompilerParams(needs_layout_passes=False),
        scratch_types=[
            pltpu.VMEM((chunk,), jnp.float32),
            pltpu.VMEM((chunk,), jnp.float32),
            pltpu.VMEM((out_c,), jnp.float32),
            pltpu.VMEM((out_c,), jnp.float32),
            pltpu.VMEM((chunk,), jnp.int32),
            pltpu.VMEM((chunk,), jnp.int32),
            pltpu.SemaphoreType.DMA,
            pltpu.SemaphoreType.DMA,
        ],
    )
    def onehot(x_hbm, out_hbm, x0, x1, buf0, buf1, idx0, idx1, sem0, sem1):
        wid = lax.axis_index("s") * NC + lax.axis_index("c")
        tile_base = wid * per_tile
        xs = (x0, x1)
        bufs = (buf0, buf1)
        idxs = (idx0, idx1)
        sems = (sem0, sem1)
        zeros = jnp.zeros((L,), jnp.float32)
        ones = jnp.ones((L,), jnp.float32)
        lane = lax.iota(jnp.int32, L)

        @pl.loop(0, out_c // L)
        def _(t):
            buf0[pl.ds(t * L, L)] = zeros
            buf1[pl.ds(t * L, L)] = zeros

        def fill_and_send(b, k):
            goff = tile_base + k * chunk
            pltpu.sync_copy(x_hbm.at[pl.ds(goff, chunk)], xs[b])

            @pl.loop(0, chunk // L)
            def _(t):
                xv = xs[b][pl.ds(t * L, L)]
                xi = jnp.clip(xv.astype(jnp.int32), 0, NUM_CLASSES - 1)
                idx = (t * L + lane) * NUM_CLASSES + xi
                idxs[b][pl.ds(t * L, L)] = idx
                plsc.store_scatter(bufs[b], [idx], ones)

            pltpu.async_copy(
                bufs[b],
                out_hbm.at[pl.ds(goff * NUM_CLASSES, out_c)],
                sems[b],
            )

        def wait_and_rezero(b):
            pltpu.make_async_copy(
                bufs[b], out_hbm.at[pl.ds(0, out_c)], sems[b]
            ).wait()

            @pl.loop(0, chunk // L)
            def _(t):
                idx = idxs[b][pl.ds(t * L, L)]
                plsc.store_scatter(bufs[b], [idx], zeros)

        fill_and_send(0, 0)
        fill_and_send(1, 1)

        @pl.loop(2, n_chunks, step=2)
        def _(k):
            for b in range(2):
                wait_and_rezero(b)
                fill_and_send(b, k + b)

        pltpu.make_async_copy(buf0, out_hbm.at[pl.ds(0, out_c)], sem0).wait()
        pltpu.make_async_copy(buf1, out_hbm.at[pl.ds(0, out_c)], sem1).wait()

    return onehot


def kernel(x):
    total = x.size
    out_flat = _make_onehot(total)(x.reshape(total))
    return out_flat.reshape(*x.shape, NUM_CLASSES)


# speedup vs baseline: 42.7625x; 9.4098x over previous
import functools

import jax
import jax.numpy as jnp
from jax import lax
from jax.experimental import pallas as pl
from jax.experimental.pallas import tpu as pltpu
from jax.experimental.pallas import tpu_sc as plsc

NUM_CLASSES = 12
NC = 2
NS = 16
L = 16
NW = NC * NS


@functools.lru_cache(maxsize=None)
def _make_planes(total: int):
    assert total % NW == 0
    per_tile = total // NW
    chunk = 3200
    while per_tile % chunk or (per_tile // chunk) % 2:
        chunk //= 2
    assert chunk % L == 0
    n_chunks = per_tile // chunk
    out_c = chunk * NUM_CLASSES

    mesh = plsc.VectorSubcoreMesh(core_axis_name="c", subcore_axis_name="s")

    @functools.partial(
        pl.kernel,
        out_type=jax.ShapeDtypeStruct((total * NUM_CLASSES,), jnp.float32),
        mesh=mesh,
        compiler_params=pltpu.CompilerParams(needs_layout_passes=False),
        scratch_types=[
            pltpu.VMEM((chunk,), jnp.float32),
            pltpu.VMEM((chunk,), jnp.float32),
            pltpu.VMEM((out_c,), jnp.float32),
            pltpu.VMEM((out_c,), jnp.float32),
            pltpu.SemaphoreType.DMA,
            pltpu.SemaphoreType.DMA,
        ],
    )
    def planes(x_hbm, out_hbm, x0, x1, buf0, buf1, sem0, sem1):
        wid = lax.axis_index("s") * NC + lax.axis_index("c")
        tile_base = wid * per_tile
        xs = (x0, x1)
        bufs = (buf0, buf1)
        sems = (sem0, sem1)
        ones = jnp.ones((L,), jnp.float32)
        zeros = jnp.zeros((L,), jnp.float32)

        def fill_and_send(b, k):
            goff = tile_base + k * chunk
            pltpu.sync_copy(x_hbm.at[pl.ds(goff, chunk)], xs[b])

            @pl.loop(0, chunk // L)
            def _(t):
                xv = xs[b][pl.ds(t * L, L)]
                for c in range(NUM_CLASSES):
                    ov = jnp.where(xv == float(c), ones, zeros)
                    bufs[b][pl.ds(c * chunk + t * L, L)] = ov

            for c in range(NUM_CLASSES):
                pltpu.async_copy(
                    bufs[b].at[pl.ds(c * chunk, chunk)],
                    out_hbm.at[pl.ds(c * total + goff, chunk)],
                    sems[b],
                )

        def drain(b):
            for c in range(NUM_CLASSES):
                pltpu.make_async_copy(
                    bufs[b].at[pl.ds(c * chunk, chunk)],
                    out_hbm.at[pl.ds(0, chunk)],
                    sems[b],
                ).wait()

        fill_and_send(0, 0)
        fill_and_send(1, 1)

        @pl.loop(2, n_chunks, step=2)
        def _(k):
            for b in range(2):
                drain(b)
                fill_and_send(b, k + b)

        drain(0)
        drain(1)

    return planes


def kernel(x):
    rows, cols = x.shape
    total = x.size
    xt_flat = x.T.reshape(total)
    out_flat = _make_planes(total)(xt_flat)
    out_t = out_flat.reshape(NUM_CLASSES, cols, rows)
    return out_t.transpose(2, 1, 0)


# speedup vs baseline: 49.7180x; 1.1627x over previous
import functools

import jax
import jax.numpy as jnp
from jax import lax
from jax.experimental import pallas as pl
from jax.experimental.pallas import tpu as pltpu
from jax.experimental.pallas import tpu_sc as plsc

NUM_CLASSES = 12
NC = 2
NS = 16
L = 16
NW = NC * NS


@functools.lru_cache(maxsize=None)
def _make_planes(total: int):
    assert total % NW == 0
    per_tile = total // NW
    chunk = 3200
    while per_tile % chunk or (per_tile // chunk) % 2 or per_tile // chunk < 4:
        chunk //= 2
    assert chunk % L == 0
    n_chunks = per_tile // chunk
    out_c = chunk * NUM_CLASSES

    mesh = plsc.VectorSubcoreMesh(core_axis_name="c", subcore_axis_name="s")

    @functools.partial(
        pl.kernel,
        out_type=jax.ShapeDtypeStruct((total * NUM_CLASSES,), jnp.float32),
        mesh=mesh,
        compiler_params=pltpu.CompilerParams(needs_layout_passes=False),
        scratch_types=[
            pltpu.VMEM((chunk,), jnp.float32),
            pltpu.VMEM((chunk,), jnp.float32),
            pltpu.VMEM((out_c,), jnp.float32),
            pltpu.VMEM((out_c,), jnp.float32),
            pltpu.SemaphoreType.DMA,
            pltpu.SemaphoreType.DMA,
            pltpu.SemaphoreType.DMA,
            pltpu.SemaphoreType.DMA,
        ],
    )
    def planes(x_hbm, out_hbm, x0, x1, buf0, buf1, so0, so1, si0, si1):
        wid = lax.axis_index("s") * NC + lax.axis_index("c")
        tile_base = wid * per_tile
        xs = (x0, x1)
        bufs = (buf0, buf1)
        osems = (so0, so1)
        isems = (si0, si1)
        ones = jnp.ones((L,), jnp.float32)
        zeros = jnp.zeros((L,), jnp.float32)

        def start_in(b, k):
            goff = tile_base + k * chunk
            pltpu.async_copy(x_hbm.at[pl.ds(goff, chunk)], xs[b], isems[b])

        def wait_in(b):
            pltpu.make_async_copy(
                x_hbm.at[pl.ds(0, chunk)], xs[b], isems[b]
            ).wait()

        def compute_and_send(b, k):
            goff = tile_base + k * chunk

            @pl.loop(0, chunk // (2 * L))
            def _(t):
                for u in range(2):
                    off = (2 * t + u) * L
                    xv = xs[b][pl.ds(off, L)]
                    for c in range(NUM_CLASSES):
                        ov = jnp.where(xv == float(c), ones, zeros)
                        bufs[b][pl.ds(c * chunk + off, L)] = ov

            for c in range(NUM_CLASSES):
                pltpu.async_copy(
                    bufs[b].at[pl.ds(c * chunk, chunk)],
                    out_hbm.at[pl.ds(c * total + goff, chunk)],
                    osems[b],
                )

        def drain(b):
            for c in range(NUM_CLASSES):
                pltpu.make_async_copy(
                    bufs[b].at[pl.ds(c * chunk, chunk)],
                    out_hbm.at[pl.ds(0, chunk)],
                    osems[b],
                ).wait()

        start_in(0, 0)
        start_in(1, 1)
        for b in range(2):
            wait_in(b)
            compute_and_send(b, b)
            if n_chunks > 2:
                start_in(b, b + 2)

        @pl.loop(2, n_chunks - 2, step=2)
        def _(k):
            for b in range(2):
                drain(b)
                wait_in(b)
                compute_and_send(b, k + b)
                start_in(b, k + b + 2)

        for b in range(2):
            drain(b)
            wait_in(b)
            compute_and_send(b, n_chunks - 2 + b)
        drain(0)
        drain(1)

    return planes


def kernel(x):
    rows, cols = x.shape
    total = x.size
    xt_flat = x.T.reshape(total)
    out_flat = _make_planes(total)(xt_flat)
    out_t = out_flat.reshape(NUM_CLASSES, cols, rows)
    return out_t.transpose(2, 1, 0)
